# Initial kernel scaffold; baseline (speedup 1.0000x reference)
import functools

import jax
import jax.numpy as jnp
from jax import lax
from jax.experimental import pallas as pl
from jax.experimental.pallas import tpu as pltpu
from jax.experimental.pallas import tpu_sc as plsc

_N = 10000
_E = 320000
_D = 128
_G = 128
_NSC = 2
_NT = 16
_NW = _NSC * _NT
_K = 128
_NP = 12288
_DUMMY = _NP - 1
_EPW = 10112
_CMSG = _EPW // _K
_EP = _NW * _EPW
_PT = 256
_PDUMMY = _PT - 1
_CPOOL = _NP // _NW // _K
_RB = 512
_NBLK = _NP // _RB

_MESH = plsc.VectorSubcoreMesh(core_axis_name="c", subcore_axis_name="s")
_F32 = jnp.float32


def _build_count(T, C):
    rpt = T // _NT
    nblk = max(1, rpt // _K)
    blk = min(_K, rpt)

    @functools.partial(
        pl.kernel, mesh=_MESH,
        out_type=jax.ShapeDtypeStruct((_NSC, T, 16), _F32),
        scratch_types=[
            pltpu.VMEM((1, _K), jnp.int32),
            pltpu.VMEM((_K, 16), _F32),
            pltpu.VMEM((_K, 16), _F32),
            pltpu.VMEM_SHARED((T, 16), _F32),
        ],
    )
    def count_kernel(idx_hbm, ones_hbm, zeros_hbm, out_hbm, iv, onesv, stv, shared):
        cid = lax.axis_index("c")
        sid = lax.axis_index("s")
        wid = cid * _NT + sid
        pltpu.sync_copy(ones_hbm, onesv)
        pltpu.sync_copy(zeros_hbm, stv)
        for k in range(nblk):
            pltpu.sync_copy(stv.at[pl.ds(0, blk)],
                            shared.at[pl.ds(sid * rpt + k * _K, blk)])
        plsc.subcore_barrier()

        @pl.loop(0, C)
        def _(c):
            pltpu.sync_copy(idx_hbm.at[wid, c], iv.at[0])
            pltpu.sync_copy(onesv, shared.at[iv.at[0]], add=True)

        plsc.subcore_barrier()
        for k in range(nblk):
            sl = pl.ds(sid * rpt + k * _K, blk)
            pltpu.sync_copy(shared.at[sl], stv.at[pl.ds(0, blk)])
            pltpu.sync_copy(stv.at[pl.ds(0, blk)], out_hbm.at[cid, sl])

    return count_kernel


def _build_msg():
    rpt = _NP // _NT
    nblk = rpt // _K

    @functools.partial(
        pl.kernel, mesh=_MESH,
        out_type=jax.ShapeDtypeStruct((_NSC, _NP, _D), _F32),
        scratch_types=[
            pltpu.VMEM((1, _K), jnp.int32),
            pltpu.VMEM((1, _K), jnp.int32),
            pltpu.VMEM((_K, _D), _F32),
            pltpu.VMEM((_K, _D), _F32),
            pltpu.VMEM_SHARED((_NP, _D), _F32),
        ],
    )
    def msg_kernel(h_hbm, src_hbm, dst_hbm, zeros_hbm, out_hbm, sv, dv, rows, stv, shared):
        cid = lax.axis_index("c")
        sid = lax.axis_index("s")
        wid = cid * _NT + sid
        pltpu.sync_copy(zeros_hbm, stv)
        for k in range(nblk):
            pltpu.sync_copy(stv, shared.at[pl.ds(sid * rpt + k * _K, _K)])
        plsc.subcore_barrier()

        @pl.loop(0, _CMSG)
        def _(c):
            pltpu.sync_copy(src_hbm.at[wid, c], sv.at[0])
            pltpu.sync_copy(dst_hbm.at[wid, c], dv.at[0])
            pltpu.sync_copy(h_hbm.at[sv.at[0]], rows)
            pltpu.sync_copy(rows, shared.at[dv.at[0]], add=True)

        plsc.subcore_barrier()
        for k in range(nblk):
            sl = pl.ds(sid * rpt + k * _K, _K)
            pltpu.sync_copy(shared.at[sl], stv)
            pltpu.sync_copy(stv, out_hbm.at[cid, sl])

    return msg_kernel


def _build_pool():
    rpt = _PT // _NT

    @functools.partial(
        pl.kernel, mesh=_MESH,
        out_type=(jax.ShapeDtypeStruct((_NSC, _PT, _D), _F32),
                  jax.ShapeDtypeStruct((_NSC, _PT, 16), _F32)),
        scratch_types=[
            pltpu.VMEM((1, _K), jnp.int32),
            pltpu.VMEM((_K, _D), _F32),
            pltpu.VMEM((_K, 16), _F32),
            pltpu.VMEM((_K, 16), _F32),
            pltpu.VMEM_SHARED((_PT, _D), _F32),
            pltpu.VMEM_SHARED((_PT, 16), _F32),
        ],
    )
    def pool_kernel(h_hbm, b_hbm, z128_hbm, z16_hbm, o16_hbm,
                    pool_hbm, cnt_hbm, iv, rows, onesv, st16, shp, shc):
        cid = lax.axis_index("c")
        sid = lax.axis_index("s")
        wid = cid * _NT + sid
        pltpu.sync_copy(z128_hbm, rows)
        pltpu.sync_copy(rows.at[pl.ds(0, rpt)], shp.at[pl.ds(sid * rpt, rpt)])
        pltpu.sync_copy(z16_hbm, st16)
        pltpu.sync_copy(st16.at[pl.ds(0, rpt)], shc.at[pl.ds(sid * rpt, rpt)])
        pltpu.sync_copy(o16_hbm, onesv)
        plsc.subcore_barrier()

        @pl.loop(0, _CPOOL)
        def _(c):
            pltpu.sync_copy(b_hbm.at[wid, c], iv.at[0])
            pltpu.sync_copy(h_hbm.at[pl.ds(wid * (_NP // _NW) + c * _K, _K)], rows)
            pltpu.sync_copy(rows, shp.at[iv.at[0]], add=True)
            pltpu.sync_copy(onesv, shc.at[iv.at[0]], add=True)

        plsc.subcore_barrier()
        sl = pl.ds(sid * rpt, rpt)
        pltpu.sync_copy(shp.at[sl], rows.at[pl.ds(0, rpt)])
        pltpu.sync_copy(rows.at[pl.ds(0, rpt)], pool_hbm.at[cid, sl])
        pltpu.sync_copy(shc.at[sl], st16.at[pl.ds(0, rpt)])
        pltpu.sync_copy(st16.at[pl.ds(0, rpt)], cnt_hbm.at[cid, sl])

    return pool_kernel


_SC_COUNT = _build_count(_NP, _CMSG)
_SC_MSG = _build_msg()
_SC_POOL = _build_pool()


def _dis_of(degp):
    return lax.rsqrt(degp[0, :, 0:1] + degp[1, :, 0:1] + 1.0)


def _tc_a_body(x_ref, degp_ref, w_ref, o_ref):
    dis = _dis_of(degp_ref[...])
    h = jnp.dot(x_ref[...], w_ref[...], precision=lax.Precision.HIGHEST)
    o_ref[...] = h * dis


def _tc_b_body(s_ref, hs_ref, degp_ref, b_ref, w_ref, o_ref):
    dis = _dis_of(degp_ref[...])
    t = (s_ref[0] + s_ref[1] + hs_ref[...]) * dis + b_ref[...]
    t = jnp.maximum(t, 0.0)
    o_ref[...] = jnp.dot(t, w_ref[...], precision=lax.Precision.HIGHEST) * dis


def _tc_c_body(s_ref, hs_ref, degp_ref, b_ref, o_ref):
    dis = _dis_of(degp_ref[...])
    t = (s_ref[0] + s_ref[1] + hs_ref[...]) * dis + b_ref[...]
    o_ref[...] = jnp.maximum(t, 0.0)


def _tc_d_body(pool_ref, cnt_ref, wl_ref, bl_ref, o_ref):
    pooled = pool_ref[0] + pool_ref[1]
    counts = cnt_ref[0, :, 0:1] + cnt_ref[1, :, 0:1]
    pooled = pooled / jnp.maximum(counts, 1.0)
    s = jnp.sum(pooled * wl_ref[...], axis=1, keepdims=True) + bl_ref[...]
    o_ref[...] = jax.nn.sigmoid(s)


_ROW = pl.BlockSpec((_RB, _D), lambda i: (i, 0))
_DEGB = pl.BlockSpec((_NSC, _RB, 16), lambda i: (0, i, 0))
_SROW = pl.BlockSpec((_NSC, _RB, _D), lambda i: (0, i, 0))
_WFULL = pl.BlockSpec((_D, _D), lambda i: (0, 0))
_BROW = pl.BlockSpec((1, _D), lambda i: (0, 0))

_TC_A = pl.pallas_call(
    _tc_a_body, grid=(_NBLK,),
    in_specs=[_ROW, _DEGB, _WFULL], out_specs=_ROW,
    out_shape=jax.ShapeDtypeStruct((_NP, _D), _F32))

_TC_B = pl.pallas_call(
    _tc_b_body, grid=(_NBLK,),
    in_specs=[_SROW, _ROW, _DEGB, _BROW, _WFULL], out_specs=_ROW,
    out_shape=jax.ShapeDtypeStruct((_NP, _D), _F32))

_TC_C = pl.pallas_call(
    _tc_c_body, grid=(_NBLK,),
    in_specs=[_SROW, _ROW, _DEGB, _BROW], out_specs=_ROW,
    out_shape=jax.ShapeDtypeStruct((_NP, _D), _F32))

_TC_D = pl.pallas_call(
    _tc_d_body,
    out_shape=jax.ShapeDtypeStruct((_PT, 1), _F32))


def kernel(x, edge_index, batch, W1, b1, W2, b2, Wlin, blin):
    src = edge_index[0]
    dst = edge_index[1]
    pad_e = _EP - _E
    srcp = jnp.concatenate(
        [src, jnp.zeros((pad_e,), jnp.int32)]).reshape(_NW, _CMSG, _K)
    dstp = jnp.concatenate(
        [dst, jnp.full((pad_e,), _DUMMY, jnp.int32)]).reshape(_NW, _CMSG, _K)
    xp = jnp.zeros((_NP, _D), _F32).at[:_N].set(x)
    batchp = jnp.concatenate(
        [batch, jnp.full((_NP - _N,), _PDUMMY, jnp.int32)]
    ).reshape(_NW, _CPOOL, _K)
    z128 = jnp.zeros((_K, _D), _F32)
    z16 = jnp.zeros((_K, 16), _F32)
    o16 = jnp.ones((_K, 16), _F32)
    b1r = b1.reshape(1, _D)
    b2r = b2.reshape(1, _D)
    wlr = Wlin.reshape(1, _D)
    blr = blin.reshape(1, 1)

    degp = _SC_COUNT(dstp, o16, z16)
    hs1 = _TC_A(xp, degp, W1)
    s1p = _SC_MSG(hs1, srcp, dstp, z128)
    hs2 = _TC_B(s1p, hs1, degp, b1r, W2)
    s2p = _SC_MSG(hs2, srcp, dstp, z128)
    h3 = _TC_C(s2p, hs2, degp, b2r)
    poolp, cntp = _SC_POOL(h3, batchp, z128, z16, o16)
    outg = _TC_D(poolp, cntp, wlr, blr)
    return outg[:_G, 0]


# speedup vs baseline: 9.7230x; 9.7230x over previous
import functools

import jax
import jax.numpy as jnp
from jax import lax
from jax.experimental import pallas as pl
from jax.experimental.pallas import tpu as pltpu
from jax.experimental.pallas import tpu_sc as plsc

_N = 10000
_E = 320000
_D = 128
_G = 128
_NSC = 2
_NT = 16
_NW = _NSC * _NT
_K = 128
_NP = 10240
_DUMMY = _NP - 1
_EPW = 10112
_CMSG = _EPW // _K
_EP = _NW * _EPW
_PT = 256
_PDUMMY = _PT - 1
_KP = 64
_CPOOL = _NP // _NW // _KP
_RB = 512
_NBLK = _NP // _RB

_F32 = jnp.float32


@functools.lru_cache(maxsize=None)
def _sc_mesh():
    return plsc.VectorSubcoreMesh(core_axis_name="c", subcore_axis_name="s",
                                  num_cores=_NSC, num_subcores=_NT)


def _build_count(T, C):
    rpt = T // _NT
    nblk = max(1, rpt // _K)
    blk = min(_K, rpt)

    @functools.partial(
        pl.kernel, mesh=_sc_mesh(),
        out_type=jax.ShapeDtypeStruct((_NSC, T, _D), _F32),
        scratch_types=[
            pltpu.VMEM((1, _K), jnp.int32),
            pltpu.VMEM((_K, _D), _F32),
            pltpu.VMEM((_K, _D), _F32),
            pltpu.VMEM_SHARED((T, _D), _F32),
        ],
    )
    def count_kernel(idx_hbm, ones_hbm, zeros_hbm, out_hbm, iv, onesv, stv, shared):
        cid = lax.axis_index("c")
        sid = lax.axis_index("s")
        wid = cid * _NT + sid
        pltpu.sync_copy(ones_hbm, onesv)
        pltpu.sync_copy(zeros_hbm, stv)
        for k in range(nblk):
            pltpu.sync_copy(stv.at[pl.ds(0, blk)],
                            shared.at[pl.ds(sid * rpt + k * _K, blk)])
        plsc.subcore_barrier()

        @pl.loop(0, C)
        def _(c):
            pltpu.sync_copy(idx_hbm.at[wid, c], iv.at[0])
            pltpu.sync_copy(onesv, shared.at[iv.at[0]], add=True)

        plsc.subcore_barrier()
        for k in range(nblk):
            sl = pl.ds(sid * rpt + k * _K, blk)
            pltpu.sync_copy(shared.at[sl], stv.at[pl.ds(0, blk)])
            pltpu.sync_copy(stv.at[pl.ds(0, blk)], out_hbm.at[cid, sl])

    return count_kernel


def _build_msg():
    rpt = _NP // _NT
    nblk = rpt // _K

    @functools.partial(
        pl.kernel, mesh=_sc_mesh(),
        out_type=jax.ShapeDtypeStruct((_NSC, _NP, _D), _F32),
        scratch_types=[
            pltpu.VMEM((1, _K), jnp.int32),
            pltpu.VMEM((1, _K), jnp.int32),
            pltpu.VMEM((_K, _D), _F32),
            pltpu.VMEM((_K, _D), _F32),
            pltpu.VMEM_SHARED((_NP, _D), _F32),
        ],
    )
    def msg_kernel(h_hbm, src_hbm, dst_hbm, zeros_hbm, out_hbm, sv, dv, rows, stv, shared):
        cid = lax.axis_index("c")
        sid = lax.axis_index("s")
        wid = cid * _NT + sid
        pltpu.sync_copy(zeros_hbm, stv)
        for k in range(nblk):
            pltpu.sync_copy(stv, shared.at[pl.ds(sid * rpt + k * _K, _K)])
        plsc.subcore_barrier()

        @pl.loop(0, _CMSG)
        def _(c):
            pltpu.sync_copy(src_hbm.at[wid, c], sv.at[0])
            pltpu.sync_copy(dst_hbm.at[wid, c], dv.at[0])
            pltpu.sync_copy(h_hbm.at[sv.at[0]], rows)
            pltpu.sync_copy(rows, shared.at[dv.at[0]], add=True)

        plsc.subcore_barrier()
        for k in range(nblk):
            sl = pl.ds(sid * rpt + k * _K, _K)
            pltpu.sync_copy(shared.at[sl], stv)
            pltpu.sync_copy(stv, out_hbm.at[cid, sl])

    return msg_kernel


def _build_pool():
    rpt = _PT // _NT

    @functools.partial(
        pl.kernel, mesh=_sc_mesh(),
        out_type=(jax.ShapeDtypeStruct((_NSC, _PT, _D), _F32),
                  jax.ShapeDtypeStruct((_NSC, _PT, _D), _F32)),
        scratch_types=[
            pltpu.VMEM((1, _KP), jnp.int32),
            pltpu.VMEM((_KP, _D), _F32),
            pltpu.VMEM((_KP, _D), _F32),
            pltpu.VMEM_SHARED((_PT, _D), _F32),
            pltpu.VMEM_SHARED((_PT, _D), _F32),
        ],
    )
    def pool_kernel(h_hbm, b_hbm, z128_hbm, o128_hbm,
                    pool_hbm, cnt_hbm, iv, rows, onesv, shp, shc):
        cid = lax.axis_index("c")
        sid = lax.axis_index("s")
        wid = cid * _NT + sid
        pltpu.sync_copy(z128_hbm.at[pl.ds(0, _KP)], rows)
        pltpu.sync_copy(rows.at[pl.ds(0, rpt)], shp.at[pl.ds(sid * rpt, rpt)])
        pltpu.sync_copy(rows.at[pl.ds(0, rpt)], shc.at[pl.ds(sid * rpt, rpt)])
        pltpu.sync_copy(o128_hbm.at[pl.ds(0, _KP)], onesv)
        plsc.subcore_barrier()

        @pl.loop(0, _CPOOL)
        def _(c):
            pltpu.sync_copy(b_hbm.at[wid, c], iv.at[0])
            pltpu.sync_copy(h_hbm.at[pl.ds(wid * (_NP // _NW) + c * _KP, _KP)], rows)
            pltpu.sync_copy(rows, shp.at[iv.at[0]], add=True)
            pltpu.sync_copy(onesv, shc.at[iv.at[0]], add=True)

        plsc.subcore_barrier()
        sl = pl.ds(sid * rpt, rpt)
        pltpu.sync_copy(shp.at[sl], rows.at[pl.ds(0, rpt)])
        pltpu.sync_copy(rows.at[pl.ds(0, rpt)], pool_hbm.at[cid, sl])
        pltpu.sync_copy(shc.at[sl], onesv.at[pl.ds(0, rpt)])
        pltpu.sync_copy(onesv.at[pl.ds(0, rpt)], cnt_hbm.at[cid, sl])

    return pool_kernel


@functools.lru_cache(maxsize=None)
def _sc_count():
    return _build_count(_NP, _CMSG)


@functools.lru_cache(maxsize=None)
def _sc_msg():
    return _build_msg()


@functools.lru_cache(maxsize=None)
def _sc_pool():
    return _build_pool()


def _dis_of(degp):
    return lax.rsqrt(degp[0, :, 0:1] + degp[1, :, 0:1] + 1.0)


def _tc_a_body(x_ref, degp_ref, w_ref, o_ref):
    dis = _dis_of(degp_ref[...])
    h = jnp.dot(x_ref[...], w_ref[...], precision=lax.Precision.HIGHEST)
    o_ref[...] = h * dis


def _tc_b_body(s_ref, hs_ref, degp_ref, b_ref, w_ref, o_ref):
    dis = _dis_of(degp_ref[...])
    t = (s_ref[0] + s_ref[1] + hs_ref[...]) * dis + b_ref[...]
    t = jnp.maximum(t, 0.0)
    o_ref[...] = jnp.dot(t, w_ref[...], precision=lax.Precision.HIGHEST) * dis


def _tc_c_body(s_ref, hs_ref, degp_ref, b_ref, o_ref):
    dis = _dis_of(degp_ref[...])
    t = (s_ref[0] + s_ref[1] + hs_ref[...]) * dis + b_ref[...]
    o_ref[...] = jnp.maximum(t, 0.0)


def _tc_d_body(pool_ref, cnt_ref, wl_ref, bl_ref, o_ref):
    pooled = pool_ref[0] + pool_ref[1]
    counts = cnt_ref[0, :, 0:1] + cnt_ref[1, :, 0:1]
    pooled = pooled / jnp.maximum(counts, 1.0)
    s = jnp.sum(pooled * wl_ref[...], axis=1, keepdims=True) + bl_ref[...]
    o_ref[...] = jax.nn.sigmoid(s)


_ROW = pl.BlockSpec((_RB, _D), lambda i: (i, 0))
_DEGB = pl.BlockSpec((_NSC, _RB, _D), lambda i: (0, i, 0))
_SROW = pl.BlockSpec((_NSC, _RB, _D), lambda i: (0, i, 0))
_WFULL = pl.BlockSpec((_D, _D), lambda i: (0, 0))
_BROW = pl.BlockSpec((1, _D), lambda i: (0, 0))

_TC_A = pl.pallas_call(
    _tc_a_body, grid=(_NBLK,),
    in_specs=[_ROW, _DEGB, _WFULL], out_specs=_ROW,
    out_shape=jax.ShapeDtypeStruct((_NP, _D), _F32))

_TC_B = pl.pallas_call(
    _tc_b_body, grid=(_NBLK,),
    in_specs=[_SROW, _ROW, _DEGB, _BROW, _WFULL], out_specs=_ROW,
    out_shape=jax.ShapeDtypeStruct((_NP, _D), _F32))

_TC_C = pl.pallas_call(
    _tc_c_body, grid=(_NBLK,),
    in_specs=[_SROW, _ROW, _DEGB, _BROW], out_specs=_ROW,
    out_shape=jax.ShapeDtypeStruct((_NP, _D), _F32))

_TC_D = pl.pallas_call(
    _tc_d_body,
    out_shape=jax.ShapeDtypeStruct((_PT, 1), _F32))


def kernel(x, edge_index, batch, W1, b1, W2, b2, Wlin, blin):
    src = edge_index[0]
    dst = edge_index[1]
    pad_e = _EP - _E
    srcp = jnp.concatenate(
        [src, jnp.zeros((pad_e,), jnp.int32)]).reshape(_NW, _CMSG, _K)
    dstp = jnp.concatenate(
        [dst, jnp.full((pad_e,), _DUMMY, jnp.int32)]).reshape(_NW, _CMSG, _K)
    xp = jnp.zeros((_NP, _D), _F32).at[:_N].set(x)
    batchp = jnp.concatenate(
        [batch, jnp.full((_NP - _N,), _PDUMMY, jnp.int32)]
    ).reshape(_NW, _CPOOL, _KP)
    z128 = jnp.zeros((_K, _D), _F32)
    o128 = jnp.ones((_K, _D), _F32)
    b1r = b1.reshape(1, _D)
    b2r = b2.reshape(1, _D)
    wlr = Wlin.reshape(1, _D)
    blr = blin.reshape(1, 1)

    degp = _sc_count()(dstp, o128, z128)
    hs1 = _TC_A(xp, degp, W1)
    s1p = _sc_msg()(hs1, srcp, dstp, z128)
    hs2 = _TC_B(s1p, hs1, degp, b1r, W2)
    s2p = _sc_msg()(hs2, srcp, dstp, z128)
    h3 = _TC_C(s2p, hs2, degp, b2r)
    poolp, cntp = _sc_pool()(h3, batchp, z128, o128)
    outg = _TC_D(poolp, cntp, wlr, blr)
    return outg[:_G, 0]
